# Initial kernel scaffold; baseline (speedup 1.0000x reference)
import jax
import jax.numpy as jnp
from jax.experimental import pallas as pl


def kernel(x, edge_index, W1_l, b1, W1_r, W2_l, b2, W2_r):
    raise NotImplementedError("write your pallas kernel here")


# speedup vs baseline: 3.1542x; 3.1542x over previous
import jax
import jax.numpy as jnp
from jax import lax
from jax.experimental import pallas as pl
from jax.experimental.pallas import tpu as pltpu
from jax.experimental.pallas import tpu_sc as plsc

N = 10000
E = 320000
D = 128

NC = 2
NS = 16
NW = NC * NS

K = 128
NCH = 80
IB = 16
NG = NCH // IB
E_PAD = NW * NCH * K
N_PAD = 10240
RPS = N_PAD // NS
RZ = K
NB = RPS // RZ


def _sc_segment_sum(with_cnt):
  out_type = [jax.ShapeDtypeStruct((NC * N_PAD, D), jnp.float32)]
  scratch = [
      pltpu.VMEM((IB, K), jnp.int32),
      pltpu.VMEM((IB, K), jnp.int32),
      pltpu.VMEM((2, K, D), jnp.float32),
      pltpu.VMEM_SHARED((N_PAD, D), jnp.float32),
      pltpu.SemaphoreType.DMA,
      pltpu.SemaphoreType.DMA,
  ]
  if with_cnt:
    out_type.append(jax.ShapeDtypeStruct((NC * N_PAD,), jnp.float32))
    scratch.append(pltpu.VMEM((K,), jnp.float32))
    scratch.append(pltpu.VMEM((RPS,), jnp.float32))
    scratch.append(pltpu.VMEM_SHARED((N_PAD,), jnp.float32))

  mesh = plsc.VectorSubcoreMesh(core_axis_name="c", subcore_axis_name="s")

  def body(f_hbm, src_hbm, dst_hbm, zacc_hbm, *outs_and_scratch):
    if with_cnt:
      (acc_out, cnt_out, srcv, dstv, rows, accsh, sem0, sem1,
       ones1d, cb1d, cnt1d) = outs_and_scratch
    else:
      acc_out, srcv, dstv, rows, accsh, sem0, sem1 = outs_and_scratch

    c = lax.axis_index("c")
    s = lax.axis_index("s")
    w = c * NS + s

    pltpu.sync_copy(zacc_hbm, rows.at[0])

    @pl.loop(0, NB)
    def _zero(t):
      pltpu.sync_copy(rows.at[0], accsh.at[pl.ds(s * RPS + t * RZ, RZ)])

    if with_cnt:
      z16 = jnp.zeros((16,), jnp.float32)
      o16 = jnp.ones((16,), jnp.float32)
      for r in range(K // 16):
        ones1d[pl.ds(16 * r, 16)] = o16
      for r in range(RPS // 16):
        cb1d[pl.ds(16 * r, 16)] = z16
      pltpu.sync_copy(cb1d, cnt1d.at[pl.ds(s * RPS, RPS)])

    plsc.subcore_barrier()

    @pl.loop(0, NG)
    def _grp(g):
      pltpu.sync_copy(src_hbm.at[pl.ds(w * NCH + g * IB, IB)], srcv)
      pltpu.sync_copy(dst_hbm.at[pl.ds(w * NCH + g * IB, IB)], dstv)

      @pl.loop(0, IB, step=2)
      def _step(j):
        cp0 = pltpu.async_copy(f_hbm.at[srcv.at[j]], rows.at[0], sem0)
        cp1 = pltpu.async_copy(f_hbm.at[srcv.at[j + 1]], rows.at[1], sem1)
        cp0.wait()
        pltpu.sync_copy(rows.at[0], accsh.at[dstv.at[j]], add=True)
        if with_cnt:
          pltpu.sync_copy(ones1d, cnt1d.at[dstv.at[j]], add=True)
        cp1.wait()
        pltpu.sync_copy(rows.at[1], accsh.at[dstv.at[j + 1]], add=True)
        if with_cnt:
          pltpu.sync_copy(ones1d, cnt1d.at[dstv.at[j + 1]], add=True)

    plsc.subcore_barrier()

    @pl.loop(0, NB)
    def _wb(t):
      pltpu.sync_copy(accsh.at[pl.ds(s * RPS + t * RZ, RZ)], rows.at[0])
      pltpu.sync_copy(rows.at[0],
                      acc_out.at[pl.ds(c * N_PAD + s * RPS + t * RZ, RZ)])

    if with_cnt:
      pltpu.sync_copy(cnt1d.at[pl.ds(s * RPS, RPS)], cb1d)
      pltpu.sync_copy(cb1d, cnt_out.at[pl.ds(c * N_PAD + s * RPS, RPS)])

  return pl.kernel(body, out_type=out_type, mesh=mesh, scratch_types=scratch)


_sc_layer1 = _sc_segment_sum(with_cnt=True)
_sc_layer2 = _sc_segment_sum(with_cnt=False)


def _tc_apply(relu):
  BLK = 1024
  grid = N_PAD // BLK

  def body(a0_ref, a1_ref, c0_ref, c1_ref, f_ref, wl_ref, wr_ref, b_ref,
           o_ref):
    cnt = jnp.maximum(c0_ref[...] + c1_ref[...], 1.0)
    agg = (a0_ref[...] + a1_ref[...]) / cnt
    y = (jnp.dot(agg, wl_ref[...], preferred_element_type=jnp.float32)
         + jnp.dot(f_ref[...], wr_ref[...], preferred_element_type=jnp.float32)
         + b_ref[...])
    o_ref[...] = jnp.maximum(y, 0.0) if relu else y

  row_spec = pl.BlockSpec((BLK, D), lambda i: (i, 0))
  cnt_spec = pl.BlockSpec((BLK, D), lambda i: (i, 0))
  w_spec = pl.BlockSpec((D, D), lambda i: (0, 0))
  b_spec = pl.BlockSpec((1, D), lambda i: (0, 0))
  return pl.pallas_call(
      body,
      grid=(grid,),
      in_specs=[row_spec, row_spec, cnt_spec, cnt_spec, row_spec, w_spec,
                w_spec, b_spec],
      out_specs=row_spec,
      out_shape=jax.ShapeDtypeStruct((N_PAD, D), jnp.float32),
  )


_tc_layer1 = _tc_apply(relu=True)
_tc_layer2 = _tc_apply(relu=False)


@jax.jit
def kernel(x, edge_index, W1_l, b1, W1_r, W2_l, b2, W2_r):
  src = edge_index[0].astype(jnp.int32)
  dst = edge_index[1].astype(jnp.int32)
  pad = E_PAD - E
  src2d = jnp.concatenate([src, jnp.zeros((pad,), jnp.int32)]).reshape(
      NW * NCH, K)
  dst2d = jnp.concatenate([dst, jnp.full((pad,), N, jnp.int32)]).reshape(
      NW * NCH, K)

  x_pad = jnp.concatenate(
      [x, jnp.zeros((N_PAD - N, D), jnp.float32)], axis=0)
  zacc = jnp.zeros((RZ, D), jnp.float32)

  acc1, cnt1 = _sc_layer1(x_pad, src2d, dst2d, zacc)
  c0 = jnp.broadcast_to(cnt1[:N_PAD, None], (N_PAD, D))
  c1 = jnp.broadcast_to(cnt1[N_PAD:, None], (N_PAD, D))
  h = _tc_layer1(acc1[:N_PAD], acc1[N_PAD:], c0, c1, x_pad,
                 W1_l, W1_r, b1.reshape(1, D))

  (acc2,) = _sc_layer2(h, src2d, dst2d, zacc)
  out = _tc_layer2(acc2[:N_PAD], acc2[N_PAD:], c0, c1, h,
                   W2_l, W2_r, b2.reshape(1, D))
  return out[:N]


# speedup vs baseline: 3.1895x; 1.0112x over previous
import jax
import jax.numpy as jnp
from jax import lax
from jax.experimental import pallas as pl
from jax.experimental.pallas import tpu as pltpu
from jax.experimental.pallas import tpu_sc as plsc

N = 10000
E = 320000
D = 128

NC = 2
NS = 16
NW = NC * NS

K = 128
NCH = 80
IB = 40
NG = NCH // IB
E_PAD = NW * NCH * K
N_PAD = 10240
RPS = N_PAD // NS
RZ = K
NB = RPS // RZ


def _sc_segment_sum(with_cnt):
  out_type = [jax.ShapeDtypeStruct((NC * N_PAD, D), jnp.float32)]
  scratch = [
      pltpu.VMEM((IB, K), jnp.int32),
      pltpu.VMEM((IB, K), jnp.int32),
      pltpu.VMEM((2, K, D), jnp.float32),
      pltpu.VMEM_SHARED((N_PAD, D), jnp.float32),
      pltpu.SemaphoreType.DMA,
      pltpu.SemaphoreType.DMA,
      pltpu.SemaphoreType.DMA,
      pltpu.SemaphoreType.DMA,
  ]
  if with_cnt:
    out_type.append(jax.ShapeDtypeStruct((NC * N_PAD,), jnp.float32))
    scratch.append(pltpu.VMEM((K,), jnp.float32))
    scratch.append(pltpu.VMEM((RPS,), jnp.float32))
    scratch.append(pltpu.VMEM_SHARED((N_PAD,), jnp.float32))

  mesh = plsc.VectorSubcoreMesh(core_axis_name="c", subcore_axis_name="s")

  def body(f_hbm, src_hbm, dst_hbm, zacc_hbm, *outs_and_scratch):
    if with_cnt:
      (acc_out, cnt_out, srcv, dstv, rows, accsh, sem0, sem1, sem2, sem3,
       ones1d, cb1d, cnt1d) = outs_and_scratch
    else:
      (acc_out, srcv, dstv, rows, accsh, sem0, sem1, sem2,
       sem3) = outs_and_scratch

    c = lax.axis_index("c")
    s = lax.axis_index("s")
    w = c * NS + s

    pltpu.sync_copy(zacc_hbm, rows.at[0])

    @pl.loop(0, NB)
    def _zero(t):
      pltpu.sync_copy(rows.at[0], accsh.at[pl.ds(s * RPS + t * RZ, RZ)])

    if with_cnt:
      z16 = jnp.zeros((16,), jnp.float32)
      o16 = jnp.ones((16,), jnp.float32)
      for r in range(K // 16):
        ones1d[pl.ds(16 * r, 16)] = o16
      for r in range(RPS // 16):
        cb1d[pl.ds(16 * r, 16)] = z16
      pltpu.sync_copy(cb1d, cnt1d.at[pl.ds(s * RPS, RPS)])

    plsc.subcore_barrier()

    @pl.loop(0, NG)
    def _grp(g):
      pltpu.sync_copy(src_hbm.at[pl.ds(w * NCH + g * IB, IB)], srcv)
      pltpu.sync_copy(dst_hbm.at[pl.ds(w * NCH + g * IB, IB)], dstv)

      @pl.loop(0, IB, step=2)
      def _step(j):
        cp0 = pltpu.async_copy(f_hbm.at[srcv.at[j]], rows.at[0], sem0)
        cp1 = pltpu.async_copy(f_hbm.at[srcv.at[j + 1]], rows.at[1], sem1)
        cp0.wait()
        sc0 = pltpu.async_copy(rows.at[0], accsh.at[dstv.at[j]], sem2,
                               add=True)
        cp1.wait()
        sc1 = pltpu.async_copy(rows.at[1], accsh.at[dstv.at[j + 1]], sem3,
                               add=True)
        if with_cnt:
          pltpu.sync_copy(ones1d, cnt1d.at[dstv.at[j]], add=True)
          pltpu.sync_copy(ones1d, cnt1d.at[dstv.at[j + 1]], add=True)
        sc0.wait()
        sc1.wait()

    plsc.subcore_barrier()

    @pl.loop(0, NB)
    def _wb(t):
      pltpu.sync_copy(accsh.at[pl.ds(s * RPS + t * RZ, RZ)], rows.at[0])
      pltpu.sync_copy(rows.at[0],
                      acc_out.at[pl.ds(c * N_PAD + s * RPS + t * RZ, RZ)])

    if with_cnt:
      pltpu.sync_copy(cnt1d.at[pl.ds(s * RPS, RPS)], cb1d)
      pltpu.sync_copy(cb1d, cnt_out.at[pl.ds(c * N_PAD + s * RPS, RPS)])

  return pl.kernel(body, out_type=out_type, mesh=mesh, scratch_types=scratch)


_sc_layer1 = _sc_segment_sum(with_cnt=True)
_sc_layer2 = _sc_segment_sum(with_cnt=False)


def _tc_apply(relu):
  BLK = 1024
  grid = N_PAD // BLK

  def body(a0_ref, a1_ref, c0_ref, c1_ref, f_ref, wl_ref, wr_ref, b_ref,
           o_ref):
    cnt = jnp.maximum(c0_ref[...] + c1_ref[...], 1.0)
    agg = (a0_ref[...] + a1_ref[...]) / cnt
    y = (jnp.dot(agg, wl_ref[...], preferred_element_type=jnp.float32)
         + jnp.dot(f_ref[...], wr_ref[...], preferred_element_type=jnp.float32)
         + b_ref[...])
    o_ref[...] = jnp.maximum(y, 0.0) if relu else y

  row_spec = pl.BlockSpec((BLK, D), lambda i: (i, 0))
  cnt_spec = pl.BlockSpec((BLK, D), lambda i: (i, 0))
  w_spec = pl.BlockSpec((D, D), lambda i: (0, 0))
  b_spec = pl.BlockSpec((1, D), lambda i: (0, 0))
  return pl.pallas_call(
      body,
      grid=(grid,),
      in_specs=[row_spec, row_spec, cnt_spec, cnt_spec, row_spec, w_spec,
                w_spec, b_spec],
      out_specs=row_spec,
      out_shape=jax.ShapeDtypeStruct((N_PAD, D), jnp.float32),
  )


_tc_layer1 = _tc_apply(relu=True)
_tc_layer2 = _tc_apply(relu=False)


@jax.jit
def kernel(x, edge_index, W1_l, b1, W1_r, W2_l, b2, W2_r):
  src = edge_index[0].astype(jnp.int32)
  dst = edge_index[1].astype(jnp.int32)
  pad = E_PAD - E
  src2d = jnp.concatenate([src, jnp.zeros((pad,), jnp.int32)]).reshape(
      NW * NCH, K)
  dst2d = jnp.concatenate([dst, jnp.full((pad,), N, jnp.int32)]).reshape(
      NW * NCH, K)

  x_pad = jnp.concatenate(
      [x, jnp.zeros((N_PAD - N, D), jnp.float32)], axis=0)
  zacc = jnp.zeros((RZ, D), jnp.float32)

  acc1, cnt1 = _sc_layer1(x_pad, src2d, dst2d, zacc)
  c0 = jnp.broadcast_to(cnt1[:N_PAD, None], (N_PAD, D))
  c1 = jnp.broadcast_to(cnt1[N_PAD:, None], (N_PAD, D))
  h = _tc_layer1(acc1[:N_PAD], acc1[N_PAD:], c0, c1, x_pad,
                 W1_l, W1_r, b1.reshape(1, D))

  (acc2,) = _sc_layer2(h, src2d, dst2d, zacc)
  out = _tc_layer2(acc2[:N_PAD], acc2[N_PAD:], c0, c1, h,
                   W2_l, W2_r, b2.reshape(1, D))
  return out[:N]


# speedup vs baseline: 3.1968x; 1.0023x over previous
import jax
import jax.numpy as jnp
from jax import lax
from jax.experimental import pallas as pl
from jax.experimental.pallas import tpu as pltpu
from jax.experimental.pallas import tpu_sc as plsc

N = 10000
E = 320000
D = 128

NC = 2
NS = 16
NW = NC * NS

K = 128
NCH = 80
IB = 40
NG = NCH // IB
E_PAD = NW * NCH * K
N_PAD = 10240
RPS = N_PAD // NS
RZ = K
NB = RPS // RZ


def _sc_segment_sum(with_cnt):
  out_type = [jax.ShapeDtypeStruct((NC * N_PAD, D), jnp.float32)]
  scratch = [
      pltpu.VMEM((IB, K), jnp.int32),
      pltpu.VMEM((IB, K), jnp.int32),
      pltpu.VMEM((2, K, D), jnp.float32),
      pltpu.VMEM_SHARED((N_PAD, D), jnp.float32),
      pltpu.SemaphoreType.DMA,
      pltpu.SemaphoreType.DMA,
      pltpu.SemaphoreType.DMA,
      pltpu.SemaphoreType.DMA,
  ]
  if with_cnt:
    out_type.append(jax.ShapeDtypeStruct((NC * N_PAD,), jnp.float32))
    scratch.append(pltpu.VMEM((K,), jnp.float32))
    scratch.append(pltpu.VMEM((RPS,), jnp.float32))
    scratch.append(pltpu.VMEM_SHARED((N_PAD,), jnp.float32))

  mesh = plsc.VectorSubcoreMesh(core_axis_name="c", subcore_axis_name="s")

  def body(f_hbm, src_hbm, dst_hbm, zacc_hbm, *outs_and_scratch):
    if with_cnt:
      (acc_out, cnt_out, srcv, dstv, rows, accsh, sem0, sem1, sem2, sem3,
       ones1d, cb1d, cnt1d) = outs_and_scratch
    else:
      (acc_out, srcv, dstv, rows, accsh, sem0, sem1, sem2,
       sem3) = outs_and_scratch

    c = lax.axis_index("c")
    s = lax.axis_index("s")
    w = c * NS + s

    pltpu.sync_copy(zacc_hbm, rows.at[0])
    zcps = [
        pltpu.async_copy(rows.at[0], accsh.at[pl.ds(s * RPS + t * RZ, RZ)],
                         sem0)
        for t in range(NB)
    ]
    for zcp in zcps:
      zcp.wait()

    if with_cnt:
      z16 = jnp.zeros((16,), jnp.float32)
      o16 = jnp.ones((16,), jnp.float32)
      for r in range(K // 16):
        ones1d[pl.ds(16 * r, 16)] = o16
      for r in range(RPS // 16):
        cb1d[pl.ds(16 * r, 16)] = z16
      pltpu.sync_copy(cb1d, cnt1d.at[pl.ds(s * RPS, RPS)])

    plsc.subcore_barrier()

    @pl.loop(0, NG)
    def _grp(g):
      pltpu.sync_copy(src_hbm.at[pl.ds(w * NCH + g * IB, IB)], srcv)
      pltpu.sync_copy(dst_hbm.at[pl.ds(w * NCH + g * IB, IB)], dstv)

      @pl.loop(0, IB, step=2)
      def _step(j):
        cp0 = pltpu.async_copy(f_hbm.at[srcv.at[j]], rows.at[0], sem0)
        cp1 = pltpu.async_copy(f_hbm.at[srcv.at[j + 1]], rows.at[1], sem1)
        cp0.wait()
        sc0 = pltpu.async_copy(rows.at[0], accsh.at[dstv.at[j]], sem2,
                               add=True)
        cp1.wait()
        sc1 = pltpu.async_copy(rows.at[1], accsh.at[dstv.at[j + 1]], sem3,
                               add=True)
        if with_cnt:
          pltpu.sync_copy(ones1d, cnt1d.at[dstv.at[j]], add=True)
          pltpu.sync_copy(ones1d, cnt1d.at[dstv.at[j + 1]], add=True)
        sc0.wait()
        sc1.wait()

    plsc.subcore_barrier()

    gsems = (sem0, sem1)
    hsems = (sem2, sem3)
    hd = [None] * NB
    for t in range(NB):
      p = t % 2
      if t >= 2:
        hd[t - 2].wait()
      gcp = pltpu.async_copy(accsh.at[pl.ds(s * RPS + t * RZ, RZ)],
                             rows.at[p], gsems[p])
      gcp.wait()
      hd[t] = pltpu.async_copy(
          rows.at[p], acc_out.at[pl.ds(c * N_PAD + s * RPS + t * RZ, RZ)],
          hsems[p])
    hd[NB - 2].wait()
    hd[NB - 1].wait()

    if with_cnt:
      pltpu.sync_copy(cnt1d.at[pl.ds(s * RPS, RPS)], cb1d)
      pltpu.sync_copy(cb1d, cnt_out.at[pl.ds(c * N_PAD + s * RPS, RPS)])

  return pl.kernel(body, out_type=out_type, mesh=mesh, scratch_types=scratch)


_sc_layer1 = _sc_segment_sum(with_cnt=True)
_sc_layer2 = _sc_segment_sum(with_cnt=False)


def _tc_apply(relu):
  BLK = 1024
  grid = N_PAD // BLK

  def body(a0_ref, a1_ref, c0_ref, c1_ref, f_ref, wl_ref, wr_ref, b_ref,
           o_ref):
    cnt = jnp.maximum(c0_ref[...] + c1_ref[...], 1.0)
    agg = (a0_ref[...] + a1_ref[...]) / cnt
    y = (jnp.dot(agg, wl_ref[...], preferred_element_type=jnp.float32)
         + jnp.dot(f_ref[...], wr_ref[...], preferred_element_type=jnp.float32)
         + b_ref[...])
    o_ref[...] = jnp.maximum(y, 0.0) if relu else y

  row_spec = pl.BlockSpec((BLK, D), lambda i: (i, 0))
  cnt_spec = pl.BlockSpec((BLK, D), lambda i: (i, 0))
  w_spec = pl.BlockSpec((D, D), lambda i: (0, 0))
  b_spec = pl.BlockSpec((1, D), lambda i: (0, 0))
  return pl.pallas_call(
      body,
      grid=(grid,),
      in_specs=[row_spec, row_spec, cnt_spec, cnt_spec, row_spec, w_spec,
                w_spec, b_spec],
      out_specs=row_spec,
      out_shape=jax.ShapeDtypeStruct((N_PAD, D), jnp.float32),
  )


_tc_layer1 = _tc_apply(relu=True)
_tc_layer2 = _tc_apply(relu=False)


@jax.jit
def kernel(x, edge_index, W1_l, b1, W1_r, W2_l, b2, W2_r):
  src = edge_index[0].astype(jnp.int32)
  dst = edge_index[1].astype(jnp.int32)
  pad = E_PAD - E
  src2d = jnp.concatenate([src, jnp.zeros((pad,), jnp.int32)]).reshape(
      NW * NCH, K)
  dst2d = jnp.concatenate([dst, jnp.full((pad,), N, jnp.int32)]).reshape(
      NW * NCH, K)

  x_pad = jnp.concatenate(
      [x, jnp.zeros((N_PAD - N, D), jnp.float32)], axis=0)
  zacc = jnp.zeros((RZ, D), jnp.float32)

  acc1, cnt1 = _sc_layer1(x_pad, src2d, dst2d, zacc)
  c0 = jnp.broadcast_to(cnt1[:N_PAD, None], (N_PAD, D))
  c1 = jnp.broadcast_to(cnt1[N_PAD:, None], (N_PAD, D))
  h = _tc_layer1(acc1[:N_PAD], acc1[N_PAD:], c0, c1, x_pad,
                 W1_l, W1_r, b1.reshape(1, D))

  (acc2,) = _sc_layer2(h, src2d, dst2d, zacc)
  out = _tc_layer2(acc2[:N_PAD], acc2[N_PAD:], c0, c1, h,
                   W2_l, W2_r, b2.reshape(1, D))
  return out[:N]
